# baseline (speedup 1.0000x reference)
import functools

import jax
import jax.numpy as jnp
import numpy as np
from jax import lax
from jax.experimental import pallas as pl
from jax.experimental.pallas import tpu as pltpu
from jax.experimental.pallas import tpu_sc as plsc

_NUM_LEVELS = 16
_MIN_RES = 16
_MAX_RES = 1024
_LOG2_HASHMAP_SIZE = 19
_TABLE = 2 ** _LOG2_HASHMAP_SIZE
_MASK = _TABLE - 1
_GROWTH = np.exp((np.log(_MAX_RES) - np.log(_MIN_RES)) / (_NUM_LEVELS - 1))
_SCALINGS = np.floor(_MIN_RES * _GROWTH ** np.arange(_NUM_LEVELS)).astype(np.float32)
_P2 = np.int32(np.uint32(2654435761))
_P3 = np.int32(805459861)

_N = 262144
_NC, _NS = 2, 16
_NW = _NC * _NS
_PW = _N // _NW
_C = 512
_CHUNKS = _PW // _C
_G16 = _C // 16


def _body(in_t, table, out, coords, offs0, offs1, idx0, idx1, rows0, rows1,
          outv, sem0, sem1):
  wid = lax.axis_index("s") * _NC + lax.axis_index("c")
  offs = (offs0, offs1)
  idxb = (idx0, idx1)
  rows = (rows0, rows1)
  sems = (sem0, sem1)
  iota = lax.iota(jnp.int32, 16)
  zeros = jnp.zeros((16,), jnp.int32)
  ones = zeros + 1

  def phase1(l, b):
    scale = float(_SCALINGS[l])
    off = l * _TABLE

    def p1(g, _):
      s = pl.multiple_of(g * 16, 16)
      xv = coords[pl.ds(s, 16)]
      yv = coords[pl.ds(pl.multiple_of(_C + s, 16), 16)]
      zv = coords[pl.ds(pl.multiple_of(2 * _C + s, 16), 16)]
      sx = xv * scale
      sy = yv * scale
      sz = zv * scale
      fxi = sx.astype(jnp.int32)
      fyi = sy.astype(jnp.int32)
      fzi = sz.astype(jnp.int32)
      offs[b][pl.ds(s, 16)] = sx - fxi.astype(jnp.float32)
      offs[b][pl.ds(pl.multiple_of(_C + s, 16), 16)] = sy - fyi.astype(jnp.float32)
      offs[b][pl.ds(pl.multiple_of(2 * _C + s, 16), 16)] = sz - fzi.astype(jnp.float32)
      ax0 = fxi
      ax1 = fxi + 1
      by0 = fyi * _P2
      by1 = by0 + _P2
      cz0 = fzi * _P3
      cz1 = cz0 + _P3
      tcc = by1 ^ cz1
      tfc = by0 ^ cz1
      tcf = by1 ^ cz0
      tff = by0 ^ cz0
      hs = (ax1 ^ tcc, ax1 ^ tfc, ax0 ^ tfc, ax0 ^ tcc,
            ax1 ^ tcf, ax1 ^ tff, ax0 ^ tff, ax0 ^ tcf)
      for c, h in enumerate(hs):
        idxb[b][pl.ds(pl.multiple_of(c * _C + s, 16), 16)] = (h & _MASK) + off
      return jnp.int32(0)

    lax.fori_loop(jnp.int32(0), jnp.int32(_G16), p1, jnp.int32(0))

  def gather_copy(b):
    return pltpu.make_async_copy(table.at[idxb[b]], rows[b], sems[b])

  def phase2(l, b):
    def p2(g, _):
      s = pl.multiple_of(g * 16, 16)
      o0 = offs[b][pl.ds(s, 16)]
      o1 = offs[b][pl.ds(pl.multiple_of(_C + s, 16), 16)]
      o2 = offs[b][pl.ds(pl.multiple_of(2 * _C + s, 16), 16)]
      c0 = 1.0 - o0
      c1 = 1.0 - o1
      c2 = 1.0 - o2
      f = []
      for c in range(8):
        rowv = iota + (c * _C + s)
        f.append((plsc.load_gather(rows[b], [rowv, zeros]),
                  plsc.load_gather(rows[b], [rowv, ones])))
      pos0 = iota * 32 + s * 32
      for feat in range(2):
        f0, f1, f2, f3 = f[0][feat], f[1][feat], f[2][feat], f[3][feat]
        f4, f5, f6, f7 = f[4][feat], f[5][feat], f[6][feat], f[7][feat]
        f03 = f0 * o0 + f3 * c0
        f12 = f1 * o0 + f2 * c0
        f56 = f5 * o0 + f6 * c0
        f47 = f4 * o0 + f7 * c0
        f0312 = f03 * o1 + f12 * c1
        f4756 = f47 * o1 + f56 * c1
        enc = f0312 * o2 + f4756 * c2
        plsc.store_scatter(outv, [pos0 + (2 * l + feat)], enc)
      return jnp.int32(0)

    lax.fori_loop(jnp.int32(0), jnp.int32(_G16), p2, jnp.int32(0))

  def chunk_body(ch, _):
    base = pl.multiple_of(wid * np.int32(_PW) + ch * np.int32(_C), _C)
    for d in range(3):
      pltpu.sync_copy(in_t.at[pl.ds(pl.multiple_of(d * _N + base, _C), _C)],
                      coords.at[pl.ds(d * _C, _C)])
    for l in range(_NUM_LEVELS):
      b = l % 2
      phase1(l, b)
      gather_copy(b).start()
      if l > 0:
        gather_copy(1 - b).wait()
        phase2(l - 1, 1 - b)
    gather_copy(1).wait()
    phase2(_NUM_LEVELS - 1, 1)
    pltpu.sync_copy(outv, out.at[pl.ds(pl.multiple_of(base * 32, _C), 32 * _C)])
    return jnp.int32(0)

  lax.fori_loop(jnp.int32(0), jnp.int32(_CHUNKS), chunk_body, jnp.int32(0))


@jax.jit
def _hash_encode(in_t, table):
  mesh = plsc.VectorSubcoreMesh(core_axis_name="c", subcore_axis_name="s",
                                num_cores=_NC, num_subcores=_NS)
  return pl.kernel(
      _body,
      out_type=jax.ShapeDtypeStruct((_N * 2 * _NUM_LEVELS,), jnp.float32),
      mesh=mesh,
      compiler_params=pltpu.CompilerParams(use_tc_tiling_on_sc=False,
                                           needs_layout_passes=False),
      scratch_types=[
          pltpu.VMEM((3 * _C,), jnp.float32),
          pltpu.VMEM((3 * _C,), jnp.float32),
          pltpu.VMEM((3 * _C,), jnp.float32),
          pltpu.VMEM((8 * _C,), jnp.int32),
          pltpu.VMEM((8 * _C,), jnp.int32),
          pltpu.VMEM((8 * _C, 2), jnp.float32),
          pltpu.VMEM((8 * _C, 2), jnp.float32),
          pltpu.VMEM((2 * _NUM_LEVELS * _C,), jnp.float32),
          pltpu.SemaphoreType.DMA,
          pltpu.SemaphoreType.DMA,
      ],
  )(in_t, table)


def kernel(in_tensor, hash_table):
  in_t = in_tensor.T.reshape(-1)
  out = _hash_encode(in_t, hash_table)
  return out.reshape(_N, 2 * _NUM_LEVELS)


# speedup vs baseline: 1.2487x; 1.2487x over previous
import jax
import jax.numpy as jnp
import numpy as np
from jax import lax
from jax.experimental import pallas as pl
from jax.experimental.pallas import tpu as pltpu
from jax.experimental.pallas import tpu_sc as plsc

_NUM_LEVELS = 16
_MIN_RES = 16
_MAX_RES = 1024
_LOG2_HASHMAP_SIZE = 19
_TABLE = 2 ** _LOG2_HASHMAP_SIZE
_MASK = _TABLE - 1
_GROWTH = np.exp((np.log(_MAX_RES) - np.log(_MIN_RES)) / (_NUM_LEVELS - 1))
_SCALINGS = np.floor(_MIN_RES * _GROWTH ** np.arange(_NUM_LEVELS)).astype(np.float32)
_P2 = np.int32(np.uint32(2654435761))
_P3 = np.int32(805459861)

_N = 262144
_NC, _NS = 2, 16
_NW = _NC * _NS
_PW = _N // _NW
_C = 512
_CHUNKS = _PW // _C
_G16 = _C // 16
_SEG = 2 * _TABLE // _NS


def _body(in_t, table, out, shared, coords, offs0, offs1, idx0, idx1,
          rows0, rows1, outv, sem0, sem1):
  wid = lax.axis_index("s") * _NC + lax.axis_index("c")
  sid = lax.axis_index("s")
  offs = (offs0, offs1)
  idxb = (idx0, idx1)
  rows = (rows0, rows1)
  sems = (sem0, sem1)
  iota = lax.iota(jnp.int32, 16)
  zeros = jnp.zeros((16,), jnp.int32)
  ones = zeros + 1

  def phase1(l, b, basev):
    scale = float(_SCALINGS[l])

    def p1(g, _):
      s = pl.multiple_of(g * 16, 16)
      xv = coords[pl.ds(s, 16)]
      yv = coords[pl.ds(pl.multiple_of(_C + s, 16), 16)]
      zv = coords[pl.ds(pl.multiple_of(2 * _C + s, 16), 16)]
      sx = xv * scale
      sy = yv * scale
      sz = zv * scale
      fxi = sx.astype(jnp.int32)
      fyi = sy.astype(jnp.int32)
      fzi = sz.astype(jnp.int32)
      offs[b][pl.ds(s, 16)] = sx - fxi.astype(jnp.float32)
      offs[b][pl.ds(pl.multiple_of(_C + s, 16), 16)] = sy - fyi.astype(jnp.float32)
      offs[b][pl.ds(pl.multiple_of(2 * _C + s, 16), 16)] = sz - fzi.astype(jnp.float32)
      ax0 = fxi
      ax1 = fxi + 1
      by0 = fyi * _P2
      by1 = by0 + _P2
      cz0 = fzi * _P3
      cz1 = cz0 + _P3
      tcc = by1 ^ cz1
      tfc = by0 ^ cz1
      tcf = by1 ^ cz0
      tff = by0 ^ cz0
      hs = (ax1 ^ tcc, ax1 ^ tfc, ax0 ^ tfc, ax0 ^ tcc,
            ax1 ^ tcf, ax1 ^ tff, ax0 ^ tff, ax0 ^ tcf)
      for c, h in enumerate(hs):
        e = (h & _MASK) * 2
        idxb[b][pl.ds(pl.multiple_of(c * _C + s, 16), 16)] = e
        idxb[b][pl.ds(pl.multiple_of(8 * _C + c * _C + s, 16), 16)] = e + 1
      return jnp.int32(0)

    pltpu.sync_copy(in_t.at[pl.ds(basev, _C)], coords.at[pl.ds(0, _C)])
    pltpu.sync_copy(in_t.at[pl.ds(pl.multiple_of(_N + basev, _C), _C)],
                    coords.at[pl.ds(_C, _C)])
    pltpu.sync_copy(in_t.at[pl.ds(pl.multiple_of(2 * _N + basev, _C), _C)],
                    coords.at[pl.ds(2 * _C, _C)])
    lax.fori_loop(jnp.int32(0), jnp.int32(_G16), p1, jnp.int32(0))

  def gather_copy(b):
    return pltpu.make_async_copy(shared.at[idxb[b]], rows[b], sems[b])

  def phase2(l, b, basev):
    def p2(g, _):
      s = pl.multiple_of(g * 16, 16)
      o0 = offs[b][pl.ds(s, 16)]
      o1 = offs[b][pl.ds(pl.multiple_of(_C + s, 16), 16)]
      o2 = offs[b][pl.ds(pl.multiple_of(2 * _C + s, 16), 16)]
      c0 = 1.0 - o0
      c1 = 1.0 - o1
      c2 = 1.0 - o2
      f = []
      for c in range(8):
        f.append((rows[b][pl.ds(pl.multiple_of(c * _C + s, 16), 16)],
                  rows[b][pl.ds(pl.multiple_of(8 * _C + c * _C + s, 16), 16)]))
      for feat in range(2):
        f0, f1, f2, f3 = f[0][feat], f[1][feat], f[2][feat], f[3][feat]
        f4, f5, f6, f7 = f[4][feat], f[5][feat], f[6][feat], f[7][feat]
        f03 = f0 * o0 + f3 * c0
        f12 = f1 * o0 + f2 * c0
        f56 = f5 * o0 + f6 * c0
        f47 = f4 * o0 + f7 * c0
        f0312 = f03 * o1 + f12 * c1
        f4756 = f47 * o1 + f56 * c1
        enc = f0312 * o2 + f4756 * c2
        outv[pl.ds(pl.multiple_of(feat * _C + s, 16), 16)] = enc
      return jnp.int32(0)

    lax.fori_loop(jnp.int32(0), jnp.int32(_G16), p2, jnp.int32(0))
    pltpu.sync_copy(outv.at[pl.ds(0, _C)],
                    out.at[pl.ds(pl.multiple_of((2 * l) * _N + basev, _C), _C)])
    pltpu.sync_copy(outv.at[pl.ds(_C, _C)],
                    out.at[pl.ds(pl.multiple_of((2 * l + 1) * _N + basev, _C), _C)])

  for l in range(_NUM_LEVELS):
    plsc.subcore_barrier()
    stage = pl.multiple_of(sid * np.int32(_SEG), _SEG)
    pltpu.sync_copy(table.at[pl.ds(stage + l * 2 * _TABLE, _SEG)],
                    shared.at[pl.ds(stage, _SEG)])
    plsc.subcore_barrier()

    def pair_body(j, _, l=l):
      base0 = pl.multiple_of(wid * np.int32(_PW) + (2 * j) * np.int32(_C), _C)
      base1 = pl.multiple_of(base0 + np.int32(_C), _C)
      phase1(l, 0, base0)
      gather_copy(0).start()
      phase1(l, 1, base1)
      gather_copy(1).start()
      gather_copy(0).wait()
      phase2(l, 0, base0)
      gather_copy(1).wait()
      phase2(l, 1, base1)
      return jnp.int32(0)

    lax.fori_loop(jnp.int32(0), jnp.int32(_CHUNKS // 2), pair_body, jnp.int32(0))


@jax.jit
def _hash_encode(in_t, table):
  mesh = plsc.VectorSubcoreMesh(core_axis_name="c", subcore_axis_name="s",
                                num_cores=_NC, num_subcores=_NS)
  return pl.kernel(
      _body,
      out_type=jax.ShapeDtypeStruct((2 * _NUM_LEVELS * _N,), jnp.float32),
      mesh=mesh,
      compiler_params=pltpu.CompilerParams(use_tc_tiling_on_sc=False,
                                           needs_layout_passes=False),
      scratch_types=[
          pltpu.VMEM_SHARED((2 * _TABLE,), jnp.float32),
          pltpu.VMEM((3 * _C,), jnp.float32),
          pltpu.VMEM((3 * _C,), jnp.float32),
          pltpu.VMEM((3 * _C,), jnp.float32),
          pltpu.VMEM((16 * _C,), jnp.int32),
          pltpu.VMEM((16 * _C,), jnp.int32),
          pltpu.VMEM((16 * _C,), jnp.float32),
          pltpu.VMEM((16 * _C,), jnp.float32),
          pltpu.VMEM((2 * _C,), jnp.float32),
          pltpu.SemaphoreType.DMA,
          pltpu.SemaphoreType.DMA,
      ],
  )(in_t, table)


def kernel(in_tensor, hash_table):
  in_t = in_tensor.T.reshape(-1)
  out = _hash_encode(in_t, hash_table.reshape(-1))
  return out.reshape(2 * _NUM_LEVELS, _N).T.reshape(_N, 2 * _NUM_LEVELS)


# speedup vs baseline: 1.2618x; 1.0105x over previous
import jax
import jax.numpy as jnp
import numpy as np
from jax import lax
from jax.experimental import pallas as pl
from jax.experimental.pallas import tpu as pltpu
from jax.experimental.pallas import tpu_sc as plsc

_NUM_LEVELS = 16
_MIN_RES = 16
_MAX_RES = 1024
_LOG2_HASHMAP_SIZE = 19
_TABLE = 2 ** _LOG2_HASHMAP_SIZE
_MASK = _TABLE - 1
_GROWTH = np.exp((np.log(_MAX_RES) - np.log(_MIN_RES)) / (_NUM_LEVELS - 1))
_SCALINGS = np.floor(_MIN_RES * _GROWTH ** np.arange(_NUM_LEVELS)).astype(np.float32)
_P2 = np.int32(np.uint32(2654435761))
_P3 = np.int32(805459861)

_N = 262144
_NC, _NS = 2, 16
_NW = _NC * _NS
_PW = _N // _NW
_C = 256
_CHUNKS = _PW // _C
_G16 = _C // 16
_SEG = 2 * _TABLE // _NS


def _body(in_t, table, out, shared, coords, offs0, offs1, idx0, idx1,
          rows0, rows1, outv, sem0, sem1):
  wid = lax.axis_index("s") * _NC + lax.axis_index("c")
  sid = lax.axis_index("s")
  offs = (offs0, offs1)
  idxb = (idx0, idx1)
  rows = (rows0, rows1)
  sems = (sem0, sem1)

  pbase = pl.multiple_of(wid * np.int32(_PW), _PW)
  for d in range(3):
    pltpu.sync_copy(in_t.at[pl.ds(pl.multiple_of(np.int32(d * _N) + pbase, _PW), _PW)],
                    coords.at[pl.ds(d * _PW, _PW)])

  def phase1(l, b, cbase):
    scale = float(_SCALINGS[l])

    def p1(g, _):
      s = pl.multiple_of(cbase + g * 16, 16)
      xv = coords[pl.ds(s, 16)]
      yv = coords[pl.ds(pl.multiple_of(_PW + s, 16), 16)]
      zv = coords[pl.ds(pl.multiple_of(2 * _PW + s, 16), 16)]
      sx = xv * scale
      sy = yv * scale
      sz = zv * scale
      fxi = sx.astype(jnp.int32)
      fyi = sy.astype(jnp.int32)
      fzi = sz.astype(jnp.int32)
      t = pl.multiple_of(g * 16, 16)
      offs[b][pl.ds(t, 16)] = sx - fxi.astype(jnp.float32)
      offs[b][pl.ds(pl.multiple_of(_C + t, 16), 16)] = sy - fyi.astype(jnp.float32)
      offs[b][pl.ds(pl.multiple_of(2 * _C + t, 16), 16)] = sz - fzi.astype(jnp.float32)
      ax0 = fxi
      ax1 = fxi + 1
      by0 = fyi * _P2
      by1 = by0 + _P2
      cz0 = fzi * _P3
      cz1 = cz0 + _P3
      tcc = by1 ^ cz1
      tfc = by0 ^ cz1
      tcf = by1 ^ cz0
      tff = by0 ^ cz0
      hs = (ax1 ^ tcc, ax1 ^ tfc, ax0 ^ tfc, ax0 ^ tcc,
            ax1 ^ tcf, ax1 ^ tff, ax0 ^ tff, ax0 ^ tcf)
      for c, h in enumerate(hs):
        e = (h & _MASK) * 2
        idxb[b][pl.ds(pl.multiple_of(c * _C + t, 16), 16)] = e
        idxb[b][pl.ds(pl.multiple_of(8 * _C + c * _C + t, 16), 16)] = e + 1
      return jnp.int32(0)

    lax.fori_loop(jnp.int32(0), jnp.int32(_G16), p1, jnp.int32(0))

  def gather_copy(b):
    return pltpu.make_async_copy(shared.at[idxb[b]], rows[b], sems[b])

  def phase2(l, b, cbase):
    def p2(g, _):
      t = pl.multiple_of(g * 16, 16)
      o0 = offs[b][pl.ds(t, 16)]
      o1 = offs[b][pl.ds(pl.multiple_of(_C + t, 16), 16)]
      o2 = offs[b][pl.ds(pl.multiple_of(2 * _C + t, 16), 16)]
      c0 = 1.0 - o0
      c1 = 1.0 - o1
      c2 = 1.0 - o2
      f = []
      for c in range(8):
        f.append((rows[b][pl.ds(pl.multiple_of(c * _C + t, 16), 16)],
                  rows[b][pl.ds(pl.multiple_of(8 * _C + c * _C + t, 16), 16)]))
      s = pl.multiple_of(cbase + g * 16, 16)
      for feat in range(2):
        f0, f1, f2, f3 = f[0][feat], f[1][feat], f[2][feat], f[3][feat]
        f4, f5, f6, f7 = f[4][feat], f[5][feat], f[6][feat], f[7][feat]
        f03 = f0 * o0 + f3 * c0
        f12 = f1 * o0 + f2 * c0
        f56 = f5 * o0 + f6 * c0
        f47 = f4 * o0 + f7 * c0
        f0312 = f03 * o1 + f12 * c1
        f4756 = f47 * o1 + f56 * c1
        enc = f0312 * o2 + f4756 * c2
        outv[pl.ds(pl.multiple_of(feat * _PW + s, 16), 16)] = enc
      return jnp.int32(0)

    lax.fori_loop(jnp.int32(0), jnp.int32(_G16), p2, jnp.int32(0))

  for l in range(_NUM_LEVELS):
    plsc.subcore_barrier()
    stage = pl.multiple_of(sid * np.int32(_SEG), _SEG)
    pltpu.sync_copy(table.at[pl.ds(stage + l * 2 * _TABLE, _SEG)],
                    shared.at[pl.ds(stage, _SEG)])
    plsc.subcore_barrier()

    def pair_body(j, _, l=l):
      cb0 = pl.multiple_of((2 * j) * np.int32(_C), _C)
      cb1 = pl.multiple_of(cb0 + np.int32(_C), _C)
      phase1(l, 0, cb0)
      gather_copy(0).start()
      phase1(l, 1, cb1)
      gather_copy(1).start()
      gather_copy(0).wait()
      phase2(l, 0, cb0)
      gather_copy(1).wait()
      phase2(l, 1, cb1)
      return jnp.int32(0)

    lax.fori_loop(jnp.int32(0), jnp.int32(_CHUNKS // 2), pair_body, jnp.int32(0))
    pltpu.sync_copy(outv.at[pl.ds(0, _PW)],
                    out.at[pl.ds(pl.multiple_of((2 * l) * _N + pbase, _PW), _PW)])
    pltpu.sync_copy(outv.at[pl.ds(_PW, _PW)],
                    out.at[pl.ds(pl.multiple_of((2 * l + 1) * _N + pbase, _PW), _PW)])


@jax.jit
def _hash_encode(in_t, table):
  mesh = plsc.VectorSubcoreMesh(core_axis_name="c", subcore_axis_name="s",
                                num_cores=_NC, num_subcores=_NS)
  return pl.kernel(
      _body,
      out_type=jax.ShapeDtypeStruct((2 * _NUM_LEVELS * _N,), jnp.float32),
      mesh=mesh,
      compiler_params=pltpu.CompilerParams(use_tc_tiling_on_sc=False,
                                           needs_layout_passes=False),
      scratch_types=[
          pltpu.VMEM_SHARED((2 * _TABLE,), jnp.float32),
          pltpu.VMEM((3 * _PW,), jnp.float32),
          pltpu.VMEM((3 * _C,), jnp.float32),
          pltpu.VMEM((3 * _C,), jnp.float32),
          pltpu.VMEM((16 * _C,), jnp.int32),
          pltpu.VMEM((16 * _C,), jnp.int32),
          pltpu.VMEM((16 * _C,), jnp.float32),
          pltpu.VMEM((16 * _C,), jnp.float32),
          pltpu.VMEM((2 * _PW,), jnp.float32),
          pltpu.SemaphoreType.DMA,
          pltpu.SemaphoreType.DMA,
      ],
  )(in_t, table)


def kernel(in_tensor, hash_table):
  in_t = in_tensor.T.reshape(-1)
  out = _hash_encode(in_t, hash_table.reshape(-1))
  return out.reshape(2 * _NUM_LEVELS, _N).T.reshape(_N, 2 * _NUM_LEVELS)
